# baseline (speedup 1.0000x reference)
import functools

import jax
import jax.numpy as jnp
from jax import lax
from jax.experimental import pallas as pl
from jax.experimental.pallas import tpu as pltpu
from jax.experimental.pallas import tpu_sc as plsc

NUM_WORKERS = 32
L = 128
NB = 4
NS = 50
SP = 2
NSTAGE = NS // SP


def _make_kernel(D):
    mesh = plsc.VectorSubcoreMesh(core_axis_name="c", subcore_axis_name="s")

    @functools.partial(
        pl.kernel,
        out_type=jax.ShapeDtypeStruct((NS, 128, L, D), jnp.float32),
        mesh=mesh,
        scratch_types=[
            pltpu.VMEM((NS + SP, NB, L), jnp.int32),
            pltpu.VMEM((2, SP, NB, L, D), jnp.float32),
            pltpu.SemaphoreType.DMA,
            pltpu.SemaphoreType.DMA,
            pltpu.SemaphoreType.DMA,
        ],
        compiler_params=pltpu.CompilerParams(use_tc_tiling_on_sc=False),
    )
    def gather_kernel(idx_hbm, table_hbm, out_hbm, idx_v, rows_v,
                      sem_g, sem_w0, sem_w1):
        wid = lax.axis_index("s") * 2 + lax.axis_index("c")
        bb0 = wid * NB
        sem_w = (sem_w0, sem_w1)

        pltpu.sync_copy(idx_hbm.at[:, pl.ds(bb0, NB)],
                        idx_v.at[pl.ds(0, NS)])
        zeros16 = jnp.zeros((16,), jnp.int32)
        for r in range(NS, NS + SP):
            for g in range(NB):
                for k in range(L // 16):
                    idx_v[r, g, pl.ds(k * 16, 16)] = zeros16

        def fire_gathers(st, par):
            for q in range(SP):
                for g in range(NB):
                    pltpu.async_copy(
                        table_hbm.at[idx_v.at[st * SP + q, g]],
                        rows_v.at[par, q, g],
                        sem_g,
                    )

        def drain_gathers(par):
            for q in range(SP):
                for g in range(NB):
                    pltpu.make_async_copy(
                        table_hbm.at[idx_v.at[0, g]],
                        rows_v.at[par, q, g],
                        sem_g,
                    ).wait()

        def writeback(st, par):
            return pltpu.make_async_copy(
                rows_v.at[par],
                out_hbm.at[pl.ds(st * SP, SP), pl.ds(bb0, NB)],
                sem_w[par],
            )

        fire_gathers(0, 0)
        drain_gathers(0)
        fire_gathers(1, 1)
        writeback(0, 0).start()
        drain_gathers(1)
        writeback(0, 0).wait()
        fire_gathers(2, 0)
        writeback(1, 1).start()

        def body(p, carry):
            for par in range(2):
                st = 2 * p + par
                drain_gathers(par)
                writeback(st - 1, 1 - par).wait()
                fire_gathers(st + 1, 1 - par)
                writeback(st, par).start()
            return carry

        lax.fori_loop(1, (NSTAGE - 1) // 2, body, 0)

        drain_gathers(0)
        writeback(NSTAGE - 2, 1).wait()
        fire_gathers(NSTAGE, 1)
        writeback(NSTAGE - 1, 0).start()

        drain_gathers(1)
        writeback(NSTAGE - 1, 0).wait()

    return gather_kernel


def kernel(token_ids, weight):
    S0, S1 = token_ids.shape
    D = weight.shape[1]
    idx = token_ids.T.reshape(S1, S0 // L, L).astype(jnp.int32)
    out = _make_kernel(D)(idx, weight)
    return out.reshape(S1, S0, D).transpose(1, 0, 2)


# speedup vs baseline: 1.1322x; 1.1322x over previous
import functools

import jax
import jax.numpy as jnp
from jax import lax
from jax.experimental import pallas as pl
from jax.experimental.pallas import tpu as pltpu
from jax.experimental.pallas import tpu_sc as plsc

NUM_WORKERS = 32
L = 128
NB = 4
NS = 50


def _make_kernel(D):
    mesh = plsc.VectorSubcoreMesh(core_axis_name="c", subcore_axis_name="s")

    @functools.partial(
        pl.kernel,
        out_type=jax.ShapeDtypeStruct((NS, 128, L, D), jnp.float32),
        mesh=mesh,
        scratch_types=[
            pltpu.VMEM((NS + 2, NB, L), jnp.int32),
            pltpu.VMEM((2, NB, L, D), jnp.float32),
            pltpu.SemaphoreType.DMA,
            pltpu.SemaphoreType.DMA,
            pltpu.SemaphoreType.DMA,
        ],
        compiler_params=pltpu.CompilerParams(use_tc_tiling_on_sc=False),
    )
    def gather_kernel(idx_hbm, table_hbm, out_hbm, idx_v, rows_v,
                      sem_g, sem_w0, sem_w1):
        wid = lax.axis_index("s") * 2 + lax.axis_index("c")
        bb0 = wid * NB
        sem_w = (sem_w0, sem_w1)

        pltpu.sync_copy(idx_hbm.at[:, pl.ds(bb0, NB)],
                        idx_v.at[pl.ds(0, NS)])
        zeros16 = jnp.zeros((16,), jnp.int32)
        for r in range(NS, NS + 2):
            for g in range(NB):
                for k in range(L // 16):
                    idx_v[r, g, pl.ds(k * 16, 16)] = zeros16

        def fire_gathers(s, par):
            for g in range(NB):
                pltpu.async_copy(
                    table_hbm.at[idx_v.at[s, g]],
                    rows_v.at[par, g],
                    sem_g,
                )

        def drain_gathers(par):
            for g in range(NB):
                pltpu.make_async_copy(
                    table_hbm.at[idx_v.at[0, g]],
                    rows_v.at[par, g],
                    sem_g,
                ).wait()

        def writeback(s, par):
            return pltpu.make_async_copy(
                rows_v.at[par],
                out_hbm.at[s, pl.ds(bb0, NB)],
                sem_w[par],
            )

        fire_gathers(0, 0)
        drain_gathers(0)
        fire_gathers(1, 1)
        writeback(0, 0).start()
        drain_gathers(1)
        writeback(0, 0).wait()
        fire_gathers(2, 0)
        writeback(1, 1).start()

        def body(p, carry):
            for par in range(2):
                s = 2 * p + par
                drain_gathers(par)
                writeback(s - 1, 1 - par).wait()
                fire_gathers(s + 1, 1 - par)
                writeback(s, par).start()
            return carry

        lax.fori_loop(1, NS // 2, body, 0)

        drain_gathers(0)
        writeback(NS - 1, 1).wait()

    return gather_kernel


def kernel(token_ids, weight):
    S0, S1 = token_ids.shape
    V, D = weight.shape
    wpad = jnp.pad(weight, ((0, 0), (0, 128 - D))).reshape(4 * V, D)
    idx = (token_ids.T.reshape(S1, S0 // L, L) * 4).astype(jnp.int32)
    out = _make_kernel(D)(idx, wpad)
    return out.reshape(S1, S0, D).transpose(1, 0, 2)
